# baseline (speedup 1.0000x reference)
import functools

import jax
import jax.numpy as jnp
from jax import lax
from jax.experimental import pallas as pl
from jax.experimental.pallas import tpu as pltpu
from jax.experimental.pallas import tpu_sc as plsc

F32 = jnp.float32

VOCAB = 16384; INNER = 64; D = 768; H = 12; DH = 64
E = 8; DFF = 2048; L = 2; B = 1; S = 2048; T = B * S
CAP = int(1.25 * B * S / E)
NROWS = E * CAP + 8
DUMP = E * CAP

_NC, _NS = 2, 16
_NW = _NC * _NS


def _ln(x, s, b):
    m = jnp.mean(x, axis=-1, keepdims=True)
    v = jnp.var(x, axis=-1, keepdims=True)
    return (x - m) / jnp.sqrt(v + 1e-5) * s + b


def _sc_gather(table, idx):
    N = idx.shape[0]
    Dd = table.shape[1]
    bpw = N // _NW
    mesh = plsc.VectorSubcoreMesh(core_axis_name="c", subcore_axis_name="s",
                                  num_cores=_NC, num_subcores=_NS)

    @functools.partial(
        pl.kernel, mesh=mesh,
        out_type=jax.ShapeDtypeStruct((N, Dd), table.dtype),
        scratch_types=[
            pltpu.VMEM((bpw,), jnp.int32),
            pltpu.VMEM((bpw, Dd), table.dtype),
            pltpu.SemaphoreType.DMA,
        ],
    )
    def k(table_hbm, idx_hbm, out_hbm, idx_v, rows_v, sem):
        wid = lax.axis_index("s") * _NC + lax.axis_index("c")
        base = wid * bpw
        pltpu.sync_copy(idx_hbm.at[pl.ds(base, bpw)], idx_v)
        pltpu.async_copy(table_hbm.at[idx_v], rows_v, sem).wait()
        pltpu.sync_copy(rows_v, out_hbm.at[pl.ds(base, bpw)])

    return k(table, idx)


def _sc_scatter(rows, idx, out_rows):
    N, Dd = rows.shape
    bpw = N // _NW
    mesh = plsc.VectorSubcoreMesh(core_axis_name="c", subcore_axis_name="s",
                                  num_cores=_NC, num_subcores=_NS)

    @functools.partial(
        pl.kernel, mesh=mesh,
        out_type=jax.ShapeDtypeStruct((out_rows, Dd), rows.dtype),
        scratch_types=[
            pltpu.VMEM((bpw,), jnp.int32),
            pltpu.VMEM((bpw, Dd), rows.dtype),
            pltpu.SemaphoreType.DMA,
        ],
    )
    def k(rows_hbm, idx_hbm, out_hbm, idx_v, rows_v, sem):
        wid = lax.axis_index("s") * _NC + lax.axis_index("c")
        base = wid * bpw
        pltpu.sync_copy(idx_hbm.at[pl.ds(base, bpw)], idx_v)
        pltpu.sync_copy(rows_hbm.at[pl.ds(base, bpw)], rows_v)
        pltpu.async_copy(rows_v, out_hbm.at[idx_v], sem).wait()

    return k(rows, idx)


def _proj(rows, proj_w):
    def body(r_ref, w_ref, o_ref):
        o_ref[...] = jnp.dot(r_ref[...], w_ref[...], preferred_element_type=F32)

    return pl.pallas_call(
        body, out_shape=jax.ShapeDtypeStruct((T, D), F32))(rows, proj_w)


def _matmul(x, w):
    N = w.shape[1]
    BN = 768

    def body(x_ref, w_ref, o_ref):
        o_ref[...] = jnp.dot(x_ref[...], w_ref[...], preferred_element_type=F32)

    return pl.pallas_call(
        body, grid=(N // BN,),
        in_specs=[
            pl.BlockSpec((T, D), lambda j: (0, 0)),
            pl.BlockSpec((D, BN), lambda j: (0, j)),
        ],
        out_specs=pl.BlockSpec((T, BN), lambda j: (0, j)),
        out_shape=jax.ShapeDtypeStruct((T, N), F32))(x, w)


def _attention(qkv3):
    RB = 1024
    NCB = S // RB

    def body(q_ref, k_ref, v_ref, o_ref):
        i = pl.program_id(1)
        q = q_ref[0]
        sc = lax.dot_general(q, k_ref[0], (((1,), (1,)), ((), ())),
                             preferred_element_type=F32) * (1.0 / 8.0)
        row = jax.lax.broadcasted_iota(jnp.int32, (RB, S), 0) + i * RB
        col = jax.lax.broadcasted_iota(jnp.int32, (RB, S), 1)
        sc = jnp.where(col <= row, sc, -1e9)
        m_run = jnp.full((RB, 1), -jnp.inf, F32)
        den = jnp.zeros((RB, 1), F32)
        acc = jnp.zeros((RB, DH), F32)
        v = v_ref[0]
        for j in range(NCB):
            sj = sc[:, j * RB:(j + 1) * RB]
            bm = jnp.max(sj, axis=-1, keepdims=True)
            m_new = jnp.maximum(m_run, bm)
            corr = jnp.where(m_run == m_new, 0.0, m_run - m_new)
            ec = jnp.exp(corr)
            u = jnp.exp(sj - m_new)
            t = ec * den
            den = t + jnp.sum(u, axis=-1, keepdims=True)
            vj = v[j * RB:(j + 1) * RB, :]
            acc = (jnp.dot(u, vj, preferred_element_type=F32) + t * acc) \
                * (1.0 / den)
            m_run = m_new
        o_ref[0] = acc

    return pl.pallas_call(
        body, grid=(H, S // RB),
        in_specs=[
            pl.BlockSpec((1, RB, DH), lambda h, i: (h, i, 0)),
            pl.BlockSpec((1, S, DH), lambda h, i: (H + h, 0, 0)),
            pl.BlockSpec((1, S, DH), lambda h, i: (2 * H + h, 0, 0)),
        ],
        out_specs=pl.BlockSpec((1, RB, DH), lambda h, i: (h, i, 0)),
        out_shape=jax.ShapeDtypeStruct((H, S, DH), F32))(qkv3, qkv3, qkv3)


def _attn_out(h, a, w):
    def body(h_ref, a_ref, w_ref, h2_ref):
        h2_ref[...] = h_ref[...] + jnp.dot(a_ref[...], w_ref[...],
                                           preferred_element_type=F32)

    return pl.pallas_call(
        body, out_shape=jax.ShapeDtypeStruct((T, D), F32))(h, a, w)


def _route(x, wr):
    def body(x_ref, wr_ref, dest_ref, gate_ref, aux_ref):
        xv = x_ref[...]
        rl = jnp.dot(xv, wr_ref[...], preferred_element_type=F32)
        m = jnp.max(rl, axis=-1, keepdims=True)
        ex = jnp.exp(rl - m)
        se = jnp.sum(ex, axis=-1, keepdims=True)
        probs = ex / se
        gate = jnp.max(probs, axis=-1, keepdims=True)
        eio = jax.lax.broadcasted_iota(jnp.int32, (T, E), 1)
        eidx = jnp.min(jnp.where(rl == m, eio, E), axis=-1, keepdims=True)
        oh = (eio == eidx).astype(F32)
        rvec = jax.lax.broadcasted_iota(jnp.int32, (T, 1), 0)
        cvec = jax.lax.broadcasted_iota(jnp.int32, (1, T), 1)
        tril = (cvec <= rvec).astype(F32)
        cnt = jnp.dot(tril, oh, preferred_element_type=F32)
        pos = jnp.sum((cnt - 1.0) * oh, axis=-1, keepdims=True).astype(jnp.int32)
        kept = pos < CAP
        dest_ref[...] = jnp.where(kept, eidx * CAP + pos, DUMP)
        gate_ref[...] = jnp.where(kept, gate, 0.0)
        lb = E * jnp.sum(jnp.mean(probs, axis=0, keepdims=True)
                         * jnp.mean(oh, axis=0, keepdims=True),
                         keepdims=True)
        lse = m + jnp.log(se)
        z = jnp.mean(lse ** 2, keepdims=True)
        lane = jax.lax.broadcasted_iota(jnp.int32, (1, 8), 1)
        aux_ref[...] = jnp.where(lane == 0, lb, z)

    return pl.pallas_call(
        body,
        out_shape=(jax.ShapeDtypeStruct((T, 1), jnp.int32),
                   jax.ShapeDtypeStruct((T, 1), F32),
                   jax.ShapeDtypeStruct((1, 8), F32)))(x, wr)


def _expert_ffn(ein, w1, w2):
    def body(x_ref, w1_ref, w2_ref, o_ref):
        hmid = jax.nn.gelu(jnp.dot(x_ref[...], w1_ref[0],
                                   preferred_element_type=F32))
        o_ref[...] = jnp.dot(hmid, w2_ref[0], preferred_element_type=F32)

    return pl.pallas_call(
        body, grid=(E,),
        in_specs=[
            pl.BlockSpec((CAP, D), lambda e: (e, 0)),
            pl.BlockSpec((1, D, DFF), lambda e: (e, 0, 0)),
            pl.BlockSpec((1, DFF, D), lambda e: (e, 0, 0)),
        ],
        out_specs=pl.BlockSpec((CAP, D), lambda e: (e, 0)),
        out_shape=jax.ShapeDtypeStruct((NROWS, D), F32))(ein, w1, w2)


def _combine(h, y, gate):
    def body(h_ref, y_ref, g_ref, o_ref):
        g = g_ref[...]
        gb = g.astype(jnp.bfloat16).astype(F32)
        yb = y_ref[...].astype(jnp.bfloat16).astype(F32)
        o_ref[...] = h_ref[...] + jnp.where(g == 0.0, 0.0, gb * yb)

    return pl.pallas_call(
        body, out_shape=jax.ShapeDtypeStruct((T, D), F32))(h, y, gate)


def _final(hn, proj_w, emb):
    VB = 2048

    def body(h_ref, pw_ref, emb_ref, o_ref):
        inner = lax.dot_general(h_ref[...], pw_ref[...], (((1,), (1,)), ((), ())),
                                preferred_element_type=F32)
        o_ref[...] = lax.dot_general(inner, emb_ref[...],
                                     (((1,), (1,)), ((), ())),
                                     preferred_element_type=F32)

    return pl.pallas_call(
        body, grid=(VOCAB // VB,),
        in_specs=[
            pl.BlockSpec((T, D), lambda j: (0, 0)),
            pl.BlockSpec((INNER, D), lambda j: (0, 0)),
            pl.BlockSpec((VB, INNER), lambda j: (j, 0)),
        ],
        out_specs=pl.BlockSpec((T, VB), lambda j: (0, j)),
        out_shape=jax.ShapeDtypeStruct((T, VOCAB), F32))(hn, proj_w, emb)


def kernel(decoder_input_ids, emb_table, proj_w, ln1_s, ln1_b, w_qkv, w_out,
           ln2_s, ln2_b, w_router, w1, w2, fn_s, fn_b):
    ids = decoder_input_ids.reshape(T).astype(jnp.int32)
    emb_pad = jnp.pad(emb_table, ((0, 0), (0, 128 - INNER)))
    proj_pad = jnp.pad(proj_w, ((0, 128 - INNER), (0, 0)))
    rows = _sc_gather(emb_pad, ids)
    h = _proj(rows, proj_pad)
    aux = jnp.float32(0.0)
    for l in range(L):
        x1 = _ln(h.reshape(B, S, D), ln1_s[l], ln1_b[l]).reshape(T, D)
        qkv = _matmul(x1, w_qkv[l])
        qkv3 = qkv.reshape(S, 3 * H, DH).transpose(1, 0, 2)
        a3 = _attention(qkv3)
        a = a3.transpose(1, 0, 2).reshape(S, D)
        h = _attn_out(h, a, w_out[l])
        x = _ln(h.reshape(B, S, D), ln2_s[l], ln2_b[l]).reshape(T, D)
        dest2, gate, auxv = _route(x, w_router[l])
        dest = dest2.reshape(T)
        ein = _sc_scatter(x, dest, NROWS)
        eo = _expert_ffn(ein, w1[l], w2[l])
        y = _sc_gather(eo, dest)
        h = _combine(h, y, gate)
        aux = aux + 0.01 * auxv[0, 0] + 0.001 * auxv[0, 1]
    logits = _final(_ln(h.reshape(B, S, D), fn_s, fn_b).reshape(T, D),
                    proj_w, emb_table)
    return logits.reshape(B, S, VOCAB), aux
